# baseline (speedup 1.0000x reference)
import jax
import jax.numpy as jnp
from jax.experimental import pallas as pl
from jax.experimental.pallas import tpu as pltpu


def _injector_body(text_ref, mask_ref, lab_ref, task_ref,
                   oe_ref, om_ref, ol_ref, sem):
    b, t, _ = task_ref.shape
    copies = []
    for i in range(b):
        copies.append(pltpu.make_async_copy(
            task_ref.at[i], oe_ref.at[i, pl.ds(0, t), :], sem))
        copies.append(pltpu.make_async_copy(
            text_ref.at[i], oe_ref.at[i, pl.ds(t, text_ref.shape[1]), :], sem))
    for c in copies:
        c.start()

    om_ref[...] = jnp.concatenate(
        [jnp.ones((b, t), dtype=om_ref.dtype), mask_ref[...]], axis=1)
    ol_ref[...] = jnp.concatenate(
        [jnp.full((b, t), -100, dtype=ol_ref.dtype), lab_ref[...]], axis=1)

    for c in copies:
        c.wait()


def kernel(text_embeds, attention_mask, labels, task_embeds):
    b, s, d = text_embeds.shape
    t = task_embeds.shape[1]
    n = t + s
    any_spec = pl.BlockSpec(memory_space=pl.ANY)
    vmem_spec = pl.BlockSpec(memory_space=pltpu.MemorySpace.VMEM)
    return pl.pallas_call(
        _injector_body,
        in_specs=[any_spec, vmem_spec, vmem_spec, any_spec],
        out_specs=[any_spec, vmem_spec, vmem_spec],
        out_shape=(
            jax.ShapeDtypeStruct((b, n, d), text_embeds.dtype),
            jax.ShapeDtypeStruct((b, n), attention_mask.dtype),
            jax.ShapeDtypeStruct((b, n), labels.dtype),
        ),
        scratch_shapes=[pltpu.SemaphoreType.DMA],
    )(text_embeds, attention_mask, labels, task_embeds)


# speedup vs baseline: 21.6123x; 21.6123x over previous
import jax
import jax.numpy as jnp
from jax.experimental import pallas as pl
from jax.experimental.pallas import tpu as pltpu


def _injector_body(text_ref, mask_ref, lab_ref, task_ref,
                   oe_ref, om_ref, ol_ref):
    i = pl.program_id(1)

    @pl.when(i == 0)
    def _copy_task():
        oe_ref[...] = task_ref[...]

    @pl.when(i > 0)
    def _copy_text():
        oe_ref[...] = text_ref[...]

    @pl.when(i == 0)
    def _masks():
        nb, t = om_ref.shape[0], task_ref.shape[1]
        om_ref[...] = jnp.concatenate(
            [jnp.ones((nb, t), dtype=om_ref.dtype), mask_ref[...]], axis=1)
        ol_ref[...] = jnp.concatenate(
            [jnp.full((nb, t), -100, dtype=ol_ref.dtype), lab_ref[...]], axis=1)


def kernel(text_embeds, attention_mask, labels, task_embeds):
    b, s, d = text_embeds.shape
    t = task_embeds.shape[1]
    n = t + s
    num_tiles = n // t
    grid = (b, num_tiles)
    full2d = lambda shape: pl.BlockSpec(shape, lambda bi, i: (0, 0))
    return pl.pallas_call(
        _injector_body,
        grid=grid,
        in_specs=[
            pl.BlockSpec((1, t, d), lambda bi, i: (bi, jnp.maximum(i - 1, 0), 0)),
            full2d((b, s)),
            full2d((b, s)),
            pl.BlockSpec((1, t, d), lambda bi, i: (bi, 0, 0)),
        ],
        out_specs=[
            pl.BlockSpec((1, t, d), lambda bi, i: (bi, i, 0)),
            full2d((b, n)),
            full2d((b, n)),
        ],
        out_shape=(
            jax.ShapeDtypeStruct((b, n, d), text_embeds.dtype),
            jax.ShapeDtypeStruct((b, n), attention_mask.dtype),
            jax.ShapeDtypeStruct((b, n), labels.dtype),
        ),
        compiler_params=pltpu.CompilerParams(
            dimension_semantics=("parallel", "arbitrary")),
    )(text_embeds, attention_mask, labels, task_embeds)


# speedup vs baseline: 21.6252x; 1.0006x over previous
import jax
import jax.numpy as jnp
from jax.experimental import pallas as pl
from jax.experimental.pallas import tpu as pltpu


def _injector_body(text_ref, mask_ref, lab_ref, task_ref,
                   oe_ref, om_ref, ol_ref):
    i = pl.program_id(1)

    @pl.when(i == 0)
    def _copy_task():
        oe_ref[...] = task_ref[...]

    @pl.when(i > 0)
    def _copy_text():
        oe_ref[...] = text_ref[...]

    @pl.when(i == 0)
    def _masks():
        nb, t = om_ref.shape[0], task_ref.shape[1]
        om_ref[...] = jnp.concatenate(
            [jnp.ones((nb, t), dtype=om_ref.dtype), mask_ref[...]], axis=1)
        ol_ref[...] = jnp.concatenate(
            [jnp.full((nb, t), -100, dtype=ol_ref.dtype), lab_ref[...]], axis=1)


def kernel(text_embeds, attention_mask, labels, task_embeds):
    b, s, d = text_embeds.shape
    t = task_embeds.shape[1]
    n = t + s
    num_tiles = n // t
    grid = (b, num_tiles)
    full2d = lambda shape: pl.BlockSpec(shape, lambda bi, i: (0, 0))
    return pl.pallas_call(
        _injector_body,
        grid=grid,
        in_specs=[
            pl.BlockSpec((1, t, d), lambda bi, i: (bi, jnp.maximum(i - 1, 0), 0)),
            full2d((b, s)),
            full2d((b, s)),
            pl.BlockSpec((1, t, d), lambda bi, i: (bi, 0, 0)),
        ],
        out_specs=[
            pl.BlockSpec((1, t, d), lambda bi, i: (bi, i, 0)),
            full2d((b, n)),
            full2d((b, n)),
        ],
        out_shape=(
            jax.ShapeDtypeStruct((b, n, d), text_embeds.dtype),
            jax.ShapeDtypeStruct((b, n), attention_mask.dtype),
            jax.ShapeDtypeStruct((b, n), labels.dtype),
        ),
        compiler_params=pltpu.CompilerParams(
            dimension_semantics=("parallel", "parallel")),
    )(text_embeds, attention_mask, labels, task_embeds)
